# baseline (speedup 1.0000x reference)
import functools

import jax
import jax.numpy as jnp
from jax import lax
from jax.experimental import pallas as pl
from jax.experimental.pallas import tpu as pltpu
from jax.experimental.pallas import tpu_sc as plsc

_NC = 2
_NS = 16
_KR = 64


def _emb_body(x_ref, y_ref, w_ref, b_ref, W_ref, bias_ref, gt_ref,
              vte_ref, vidx_ref, *, L, DX, DY, TD, DM):
    b = pl.program_id(0)
    xt = jnp.transpose(x_ref[:, b, :], (1, 0))
    yt = jnp.transpose(y_ref[0], (1, 0))

    acc = jnp.zeros((L, DM), jnp.float32)
    lane = lax.broadcasted_iota(jnp.int32, (1, TD), 1)
    for j in range(DX + 1):
        if j < DX:
            col = xt[:, j:j + 1]
            col = jnp.where(jnp.isnan(col), 0.0, col)
        else:
            col = (lax.broadcasted_iota(jnp.int32, (L, 1), 0)
                   .astype(jnp.float32) * (1.0 / L))
        aff = col * w_ref[j:j + 1, :] + b_ref[j:j + 1, :]
        tj = jnp.where(lane == 0, aff, jnp.sin(aff))
        acc = acc + jnp.dot(tj, W_ref[1 + j * TD:1 + (j + 1) * TD, :],
                            preferred_element_type=jnp.float32)
    t_val = acc + bias_ref[...] + gt_ref[1:2, :]

    w0 = W_ref[0:1, :]
    delta = gt_ref[0:1, :] - gt_ref[1:2, :]
    for d in range(DY):
        yc = yt[:, d:d + 1]
        m = jnp.isnan(yc)
        yclean = jnp.where(m, 0.0, yc)
        vte_ref[0, d] = (yclean * w0 + t_val
                         + m.astype(jnp.float32) * delta)

    @pl.when(b % 8 == 0)
    def _fill_idx():
        flat = lax.broadcasted_iota(jnp.int32, (8, DY * L), 1)
        if L & (L - 1) == 0:
            vidx_ref[...] = lax.shift_right_logical(flat, (L - 1).bit_length())
        else:
            vidx_ref[...] = flat // L


def _var_emb_sc(table_hbm, out_hbm, idx_v, buf_v, sem, *, B, L, DM):
    d = lax.axis_index("s") * _NC + lax.axis_index("c")
    for k in range(_KR // 16):
        idx_v[pl.ds(k * 16, 16)] = jnp.full((16,), d, dtype=jnp.int32)
    pltpu.async_copy(table_hbm.at[idx_v], buf_v, sem).wait()

    nchunk = L // _KR

    def _issue(i, carry):
        b = i // nchunk
        h = i - b * nchunk
        pltpu.async_copy(
            buf_v, out_hbm.at[b, d, pl.ds(h * _KR, _KR)], sem)
        return carry

    def _drain(i, carry):
        b = i // nchunk
        h = i - b * nchunk
        pltpu.make_async_copy(
            buf_v, out_hbm.at[b, d, pl.ds(h * _KR, _KR)], sem).wait()
        return carry

    lax.fori_loop(0, B * nchunk, _issue, 0)
    lax.fori_loop(0, B * nchunk, _drain, 0)


def kernel(x, y, t2v_w, t2v_b, y_emb_W, y_emb_b, var_table, given_table):
    B, L, DX = x.shape
    DY = y.shape[2]
    DM = y_emb_W.shape[1]
    TD = t2v_w.shape[1]
    assert DY == _NC * _NS and L % _KR == 0 and DM % 16 == 0

    bias2 = y_emb_b.reshape(1, DM)
    x3 = jnp.transpose(x, (2, 0, 1))
    y3 = jnp.transpose(y, (0, 2, 1))

    sc_body = functools.partial(_var_emb_sc, B=B, L=L, DM=DM)
    vemb = pl.kernel(
        sc_body,
        mesh=plsc.VectorSubcoreMesh(core_axis_name="c", subcore_axis_name="s"),
        out_type=jax.ShapeDtypeStruct((B, DY, L, DM), jnp.float32),
        scratch_types=[
            pltpu.VMEM((_KR,), jnp.int32),
            pltpu.VMEM((_KR, DM), jnp.float32),
            pltpu.SemaphoreType.DMA,
        ],
    )(var_table)

    body = functools.partial(_emb_body, L=L, DX=DX, DY=DY, TD=TD, DM=DM)
    vte, vidx = pl.pallas_call(
        body,
        grid=(B,),
        in_specs=[
            pl.BlockSpec((DX, B, L), lambda b: (0, 0, 0)),
            pl.BlockSpec((1, DY, L), lambda b: (b, 0, 0)),
            pl.BlockSpec((DX + 1, TD), lambda b: (0, 0)),
            pl.BlockSpec((DX + 1, TD), lambda b: (0, 0)),
            pl.BlockSpec((1 + (DX + 1) * TD, DM), lambda b: (0, 0)),
            pl.BlockSpec((1, DM), lambda b: (0, 0)),
            pl.BlockSpec((2, DM), lambda b: (0, 0)),
        ],
        out_specs=[
            pl.BlockSpec((1, DY, L, DM), lambda b: (b, 0, 0, 0)),
            pl.BlockSpec((8, DY * L), lambda b: (b // 8, 0)),
        ],
        out_shape=[
            jax.ShapeDtypeStruct((B, DY, L, DM), jnp.float32),
            jax.ShapeDtypeStruct((B, DY * L), jnp.int32),
        ],
        compiler_params=pltpu.CompilerParams(
            dimension_semantics=("arbitrary",)),
    )(x3, y3, t2v_w, t2v_b, y_emb_W, bias2, given_table)

    return (vte.reshape(B, DY * L, DM),
            vemb.reshape(B, DY * L, DM),
            vidx)


# speedup vs baseline: 1.0038x; 1.0038x over previous
import functools

import jax
import jax.numpy as jnp
from jax import lax
from jax.experimental import pallas as pl
from jax.experimental.pallas import tpu as pltpu
from jax.experimental.pallas import tpu_sc as plsc

_NC = 2
_NS = 16
_KR = 128


def _emb_body(x_ref, y_ref, w_ref, b_ref, W_ref, bias_ref, gt_ref,
              vte_ref, vidx_ref, *, L, DX, DY, TD, DM):
    b = pl.program_id(0)
    xt = jnp.transpose(x_ref[:, b, :], (1, 0))
    yt = jnp.transpose(y_ref[0], (1, 0))

    acc = jnp.zeros((L, DM), jnp.float32)
    lane = lax.broadcasted_iota(jnp.int32, (1, TD), 1)
    for j in range(DX + 1):
        if j < DX:
            col = xt[:, j:j + 1]
            col = jnp.where(jnp.isnan(col), 0.0, col)
        else:
            col = (lax.broadcasted_iota(jnp.int32, (L, 1), 0)
                   .astype(jnp.float32) * (1.0 / L))
        aff = col * w_ref[j:j + 1, :] + b_ref[j:j + 1, :]
        tj = jnp.where(lane == 0, aff, jnp.sin(aff))
        acc = acc + jnp.dot(tj, W_ref[1 + j * TD:1 + (j + 1) * TD, :],
                            preferred_element_type=jnp.float32)
    t_val = acc + bias_ref[...] + gt_ref[1:2, :]

    w0 = W_ref[0:1, :]
    delta = gt_ref[0:1, :] - gt_ref[1:2, :]
    for d in range(DY):
        yc = yt[:, d:d + 1]
        m = jnp.isnan(yc)
        yclean = jnp.where(m, 0.0, yc)
        vte_ref[0, d] = (yclean * w0 + t_val
                         + m.astype(jnp.float32) * delta)

    @pl.when(b % 8 == 0)
    def _fill_idx():
        flat = lax.broadcasted_iota(jnp.int32, (8, DY * L), 1)
        if L & (L - 1) == 0:
            vidx_ref[...] = lax.shift_right_logical(flat, (L - 1).bit_length())
        else:
            vidx_ref[...] = flat // L


def _var_emb_sc(table_hbm, out_hbm, idx_v, buf_v, sem, *, B, L, DM):
    d = lax.axis_index("s") * _NC + lax.axis_index("c")
    for k in range(_KR // 16):
        idx_v[pl.ds(k * 16, 16)] = jnp.full((16,), d, dtype=jnp.int32)
    pltpu.async_copy(table_hbm.at[idx_v], buf_v, sem).wait()

    nchunk = L // _KR

    def _issue(i, carry):
        b = i // nchunk
        h = i - b * nchunk
        pltpu.async_copy(
            buf_v, out_hbm.at[b, d, pl.ds(h * _KR, _KR)], sem)
        return carry

    def _drain(i, carry):
        b = i // nchunk
        h = i - b * nchunk
        pltpu.make_async_copy(
            buf_v, out_hbm.at[b, d, pl.ds(h * _KR, _KR)], sem).wait()
        return carry

    lax.fori_loop(0, B * nchunk, _issue, 0)
    lax.fori_loop(0, B * nchunk, _drain, 0)


def kernel(x, y, t2v_w, t2v_b, y_emb_W, y_emb_b, var_table, given_table):
    B, L, DX = x.shape
    DY = y.shape[2]
    DM = y_emb_W.shape[1]
    TD = t2v_w.shape[1]
    assert DY == _NC * _NS and L % _KR == 0 and DM % 16 == 0

    bias2 = y_emb_b.reshape(1, DM)
    x3 = jnp.transpose(x, (2, 0, 1))
    y3 = jnp.transpose(y, (0, 2, 1))

    sc_body = functools.partial(_var_emb_sc, B=B, L=L, DM=DM)
    vemb = pl.kernel(
        sc_body,
        mesh=plsc.VectorSubcoreMesh(core_axis_name="c", subcore_axis_name="s"),
        out_type=jax.ShapeDtypeStruct((B, DY, L, DM), jnp.float32),
        scratch_types=[
            pltpu.VMEM((_KR,), jnp.int32),
            pltpu.VMEM((_KR, DM), jnp.float32),
            pltpu.SemaphoreType.DMA,
        ],
    )(var_table)

    body = functools.partial(_emb_body, L=L, DX=DX, DY=DY, TD=TD, DM=DM)
    vte, vidx = pl.pallas_call(
        body,
        grid=(B,),
        in_specs=[
            pl.BlockSpec((DX, B, L), lambda b: (0, 0, 0)),
            pl.BlockSpec((1, DY, L), lambda b: (b, 0, 0)),
            pl.BlockSpec((DX + 1, TD), lambda b: (0, 0)),
            pl.BlockSpec((DX + 1, TD), lambda b: (0, 0)),
            pl.BlockSpec((1 + (DX + 1) * TD, DM), lambda b: (0, 0)),
            pl.BlockSpec((1, DM), lambda b: (0, 0)),
            pl.BlockSpec((2, DM), lambda b: (0, 0)),
        ],
        out_specs=[
            pl.BlockSpec((1, DY, L, DM), lambda b: (b, 0, 0, 0)),
            pl.BlockSpec((8, DY * L), lambda b: (b // 8, 0)),
        ],
        out_shape=[
            jax.ShapeDtypeStruct((B, DY, L, DM), jnp.float32),
            jax.ShapeDtypeStruct((B, DY * L), jnp.int32),
        ],
        compiler_params=pltpu.CompilerParams(
            dimension_semantics=("arbitrary",)),
    )(x3, y3, t2v_w, t2v_b, y_emb_W, bias2, given_table)

    return (vte.reshape(B, DY * L, DM),
            vemb.reshape(B, DY * L, DM),
            vidx)
